# baseline (speedup 1.0000x reference)
import functools

import jax
import jax.numpy as jnp
from jax import lax
from jax.experimental import pallas as pl
from jax.experimental.pallas import tpu as pltpu
from jax.experimental.pallas import tpu_sc as plsc

_NC = 2
_NS = 16
_CH = 128
_CHUNKS = 4
_BLK = 2048


def _sc_gather(nodes, table, betweenness, closeness, chunk, n_chunks):
  B = nodes.shape[0] // n_chunks
  row0 = chunk * B
  D = table.shape[1]
  nw = _NC * _NS
  b_per_w = B // nw
  n_ch = b_per_w // _CH

  mesh = plsc.VectorSubcoreMesh(core_axis_name="c", subcore_axis_name="s")

  @functools.partial(
      pl.kernel,
      mesh=mesh,
      out_type=(
          jax.ShapeDtypeStruct((B, D), jnp.float32),
          jax.ShapeDtypeStruct((2, B), jnp.float32),
      ),
      scratch_types=[
          pltpu.VMEM((n_ch, _CH), jnp.int32),
          pltpu.VMEM((b_per_w, D), jnp.float32),
          pltpu.VMEM((b_per_w,), jnp.float32),
          pltpu.VMEM((b_per_w,), jnp.float32),
          pltpu.SemaphoreType.DMA,
          pltpu.SemaphoreType.DMA,
      ],
  )
  def gather_kernel(nodes_hbm, table_hbm, bw_hbm, cl_hbm,
                    feats_out, bwcl_out,
                    idx_v, rows_v, bw_v, cl_v, sem, wsem):
    wid = lax.axis_index("s") * _NC + lax.axis_index("c")
    base = wid * b_per_w
    for j in range(n_ch):
      pltpu.sync_copy(nodes_hbm.at[pl.ds(row0 + base + j * _CH, _CH)],
                      idx_v.at[j])
    copies = []
    for j in range(n_ch):
      idx_j = idx_v.at[j]
      copies.append(pltpu.async_copy(
          table_hbm.at[idx_j], rows_v.at[pl.ds(j * _CH, _CH)], sem))
      copies.append(pltpu.async_copy(
          bw_hbm.at[idx_j], bw_v.at[pl.ds(j * _CH, _CH)], sem))
      copies.append(pltpu.async_copy(
          cl_hbm.at[idx_j], cl_v.at[pl.ds(j * _CH, _CH)], sem))
    writes = []
    for j in range(n_ch):
      copies[3 * j].wait()
      copies[3 * j + 1].wait()
      copies[3 * j + 2].wait()
      writes.append(pltpu.async_copy(
          rows_v.at[pl.ds(j * _CH, _CH)],
          feats_out.at[pl.ds(base + j * _CH, _CH)], wsem))
    writes.append(pltpu.async_copy(
        bw_v, bwcl_out.at[0, pl.ds(base, b_per_w)], wsem))
    writes.append(pltpu.async_copy(
        cl_v, bwcl_out.at[1, pl.ds(base, b_per_w)], wsem))
    for w in writes:
      w.wait()

  return gather_kernel(nodes, table, betweenness, closeness)


def _combine_body(feats_ref, bwcl_ref, wfc_ref, wce_ref,
                  bfc_ref, bce_ref, out_ref):
  acc = lax.dot_general(
      feats_ref[...], wfc_ref[...],
      dimension_numbers=(((1,), (1,)), ((), ())),
      preferred_element_type=jnp.float32,
  )
  cent = lax.dot_general(
      bwcl_ref[...], wce_ref[...],
      dimension_numbers=(((0,), (1,)), ((), ())),
      preferred_element_type=jnp.float32,
  )
  out_ref[...] = acc + cent + bfc_ref[...] + bce_ref[...]


def _tc_combine_chunk(acc, feats, bwcl, W_fc, W_ce, b_fc, b_ce, row0, B):
  Bc, D = feats.shape
  grid = (Bc // _BLK,)
  blk0 = row0 // _BLK
  in_specs = [
      pl.BlockSpec((_BLK, D), lambda i: (i, 0)),
      pl.BlockSpec((2, _BLK), lambda i: (0, i)),
      pl.BlockSpec((D, D), lambda i: (0, 0)),
      pl.BlockSpec((D, 2), lambda i: (0, 0)),
      pl.BlockSpec((1, D), lambda i: (0, 0)),
      pl.BlockSpec((1, D), lambda i: (0, 0)),
  ]
  args = [feats, bwcl, W_fc, W_ce, b_fc, b_ce]
  body = _combine_body
  aliases = {}
  if acc is not None:
    in_specs.insert(0, pl.BlockSpec((8, D), lambda i: (0, 0)))
    args.insert(0, acc)
    aliases = {0: 0}
    body = lambda a, *rest: _combine_body(*rest)
  return pl.pallas_call(
      body,
      grid=grid,
      in_specs=in_specs,
      out_specs=pl.BlockSpec((_BLK, D), lambda i: (blk0 + i, 0)),
      out_shape=jax.ShapeDtypeStruct((B, D), jnp.float32),
      input_output_aliases=aliases,
  )(*args)


def kernel(nodes, node_feat_table, betweenness, closeness,
           W_fc, b_fc, W_ce, b_ce):
  nodes = nodes.astype(jnp.int32)
  B = nodes.shape[0]
  Bc = B // _CHUNKS
  b_fc2 = b_fc.reshape(1, -1)
  b_ce2 = b_ce.reshape(1, -1)

  gathered = []
  for c in range(_CHUNKS):
    gathered.append(_sc_gather(nodes, node_feat_table, betweenness,
                               closeness, c, _CHUNKS))

  acc = None
  for c in range(_CHUNKS):
    feats, bwcl = gathered[c]
    acc = _tc_combine_chunk(acc, feats, bwcl, W_fc, W_ce, b_fc2, b_ce2,
                            c * Bc, B)
  return acc


# speedup vs baseline: 1.1797x; 1.1797x over previous
import functools

import jax
import jax.numpy as jnp
from jax import lax
from jax.experimental import pallas as pl
from jax.experimental.pallas import tpu as pltpu
from jax.experimental.pallas import tpu_sc as plsc

_NC = 2
_NS = 16
_CH = 128
_SPLITS = (12288, 4096)
_BLK = 2048


def _sc_gather(nodes, table, betweenness, closeness, row0, B):
  D = table.shape[1]
  nw = _NC * _NS
  b_per_w = B // nw
  n_ch = b_per_w // _CH

  mesh = plsc.VectorSubcoreMesh(core_axis_name="c", subcore_axis_name="s")

  @functools.partial(
      pl.kernel,
      mesh=mesh,
      out_type=(
          jax.ShapeDtypeStruct((B, D), jnp.float32),
          jax.ShapeDtypeStruct((2, B), jnp.float32),
      ),
      scratch_types=[
          pltpu.VMEM((n_ch, _CH), jnp.int32),
          pltpu.VMEM((b_per_w, D), jnp.float32),
          pltpu.VMEM((b_per_w,), jnp.float32),
          pltpu.VMEM((b_per_w,), jnp.float32),
          pltpu.SemaphoreType.DMA,
          pltpu.SemaphoreType.DMA,
      ],
  )
  def gather_kernel(nodes_hbm, table_hbm, bw_hbm, cl_hbm,
                    feats_out, bwcl_out,
                    idx_v, rows_v, bw_v, cl_v, sem, wsem):
    wid = lax.axis_index("s") * _NC + lax.axis_index("c")
    base = wid * b_per_w
    for j in range(n_ch):
      pltpu.sync_copy(nodes_hbm.at[pl.ds(row0 + base + j * _CH, _CH)],
                      idx_v.at[j])
    copies = []
    for j in range(n_ch):
      idx_j = idx_v.at[j]
      copies.append(pltpu.async_copy(
          table_hbm.at[idx_j], rows_v.at[pl.ds(j * _CH, _CH)], sem))
      copies.append(pltpu.async_copy(
          bw_hbm.at[idx_j], bw_v.at[pl.ds(j * _CH, _CH)], sem))
      copies.append(pltpu.async_copy(
          cl_hbm.at[idx_j], cl_v.at[pl.ds(j * _CH, _CH)], sem))
    writes = []
    for j in range(n_ch):
      copies[3 * j].wait()
      copies[3 * j + 1].wait()
      copies[3 * j + 2].wait()
      writes.append(pltpu.async_copy(
          rows_v.at[pl.ds(j * _CH, _CH)],
          feats_out.at[pl.ds(base + j * _CH, _CH)], wsem))
    writes.append(pltpu.async_copy(
        bw_v, bwcl_out.at[0, pl.ds(base, b_per_w)], wsem))
    writes.append(pltpu.async_copy(
        cl_v, bwcl_out.at[1, pl.ds(base, b_per_w)], wsem))
    for w in writes:
      w.wait()

  return gather_kernel(nodes, table, betweenness, closeness)


def _combine_body(feats_ref, bwcl_ref, wfc_ref, wce_ref,
                  bfc_ref, bce_ref, out_ref):
  acc = lax.dot_general(
      feats_ref[...], wfc_ref[...],
      dimension_numbers=(((1,), (1,)), ((), ())),
      preferred_element_type=jnp.float32,
  )
  cent = lax.dot_general(
      bwcl_ref[...], wce_ref[...],
      dimension_numbers=(((0,), (1,)), ((), ())),
      preferred_element_type=jnp.float32,
  )
  out_ref[...] = acc + cent + bfc_ref[...] + bce_ref[...]


def _tc_combine_chunk(acc, feats, bwcl, W_fc, W_ce, b_fc, b_ce, row0, B):
  Bc, D = feats.shape
  grid = (Bc // _BLK,)
  blk0 = row0 // _BLK
  in_specs = [
      pl.BlockSpec((_BLK, D), lambda i: (i, 0)),
      pl.BlockSpec((2, _BLK), lambda i: (0, i)),
      pl.BlockSpec((D, D), lambda i: (0, 0)),
      pl.BlockSpec((D, 2), lambda i: (0, 0)),
      pl.BlockSpec((1, D), lambda i: (0, 0)),
      pl.BlockSpec((1, D), lambda i: (0, 0)),
  ]
  args = [feats, bwcl, W_fc, W_ce, b_fc, b_ce]
  body = _combine_body
  aliases = {}
  if acc is not None:
    in_specs.insert(0, pl.BlockSpec((8, D), lambda i: (0, 0)))
    args.insert(0, acc)
    aliases = {0: 0}
    body = lambda a, *rest: _combine_body(*rest)
  return pl.pallas_call(
      body,
      grid=grid,
      in_specs=in_specs,
      out_specs=pl.BlockSpec((_BLK, D), lambda i: (blk0 + i, 0)),
      out_shape=jax.ShapeDtypeStruct((B, D), jnp.float32),
      input_output_aliases=aliases,
  )(*args)


def kernel(nodes, node_feat_table, betweenness, closeness,
           W_fc, b_fc, W_ce, b_ce):
  nodes = nodes.astype(jnp.int32)
  B = nodes.shape[0]
  b_fc2 = b_fc.reshape(1, -1)
  b_ce2 = b_ce.reshape(1, -1)

  row0s = [sum(_SPLITS[:c]) for c in range(len(_SPLITS))]
  gathered = []
  for c, bc in enumerate(_SPLITS):
    gathered.append(_sc_gather(nodes, node_feat_table, betweenness,
                               closeness, row0s[c], bc))

  acc = None
  for c, bc in enumerate(_SPLITS):
    feats, bwcl = gathered[c]
    acc = _tc_combine_chunk(acc, feats, bwcl, W_fc, W_ce, b_fc2, b_ce2,
                            row0s[c], B)
  return acc
